# Initial kernel scaffold; baseline (speedup 1.0000x reference)
import jax
import jax.numpy as jnp
from jax.experimental import pallas as pl


def kernel(user_ids, item_ids, genre_ids, user_table, item_table, genre_table, attn_w, attn_b, W0, b0, gamma0, beta0, W1, b1, gamma1, beta1, W2, b2, gamma2, beta2, Wp, bp):
    raise NotImplementedError("write your pallas kernel here")


# speedup vs baseline: 5.7167x; 5.7167x over previous
import functools

import jax
import jax.numpy as jnp
from jax import lax
from jax.experimental import pallas as pl
from jax.experimental.pallas import tpu as pltpu
from jax.experimental.pallas import tpu_sc as plsc

_EPS = 1e-5

_NC = 2
_NS = 16
_NW = _NC * _NS
_SUB = 128


def _sc_gather(user_ids, item_ids, user_table, item_table):
    B = user_ids.shape[0]
    D = user_table.shape[1]
    ch = B // _NW
    nchunk = ch // _SUB

    uids2 = user_ids.reshape(B // _SUB, _SUB)
    iids2 = item_ids.reshape(B // _SUB, _SUB)

    mesh = plsc.VectorSubcoreMesh(core_axis_name="c", subcore_axis_name="s")

    @functools.partial(
        pl.kernel,
        mesh=mesh,
        out_type=(
            jax.ShapeDtypeStruct((B, D), jnp.float32),
            jax.ShapeDtypeStruct((B, D), jnp.float32),
        ),
        scratch_types=[
            pltpu.VMEM((nchunk, _SUB), jnp.int32),
            pltpu.VMEM((ch, D), jnp.float32),
            pltpu.SemaphoreType.DMA,
        ],
    )
    def gk(uids, iids, ut, it, ue_out, ie_out, idx_v, rows_v, sem):
        wid = lax.axis_index("s") * _NC + lax.axis_index("c")
        base = wid * ch
        for ids, tbl, out in ((uids, ut, ue_out), (iids, it, ie_out)):
            pltpu.sync_copy(ids.at[pl.ds(wid * nchunk, nchunk)], idx_v)
            handles = []
            for j in range(nchunk):
                handles.append(
                    pltpu.async_copy(
                        tbl.at[idx_v.at[j]],
                        rows_v.at[pl.ds(j * _SUB, _SUB)],
                        sem,
                    )
                )
            for h in handles:
                h.wait()
            pltpu.sync_copy(rows_v, out.at[pl.ds(base, ch)])

    return gk(uids2, iids2, user_table, item_table)


def _p0_body(gid_ref, ue_ref, ie_ref, gt_ref, aw_ref, w_ref, b_ref,
             z_ref, s_ref, ss_ref, *, G, NGP):
    gid = gid_ref[...]
    gt = gt_ref[...]
    aw = aw_ref[...]
    lt = jnp.sum(gt * aw, axis=1)
    tb = gid.shape[0]
    iota_t = lax.broadcasted_iota(jnp.int32, (tb, NGP), 1)
    ohs, ls = [], []
    for g in range(G):
        oh = (gid[:, g:g + 1] == iota_t).astype(jnp.float32)
        ohs.append(oh)
        ls.append(jnp.sum(oh * lt[None, :], axis=1, keepdims=True))
    m = ls[0]
    for g in range(1, G):
        m = jnp.maximum(m, ls[g])
    es = [jnp.exp(l - m) for l in ls]
    denom = es[0]
    for g in range(1, G):
        denom = denom + es[g]
    wsum = es[0] * ohs[0]
    for g in range(1, G):
        wsum = wsum + es[g] * ohs[g]
    wsum = wsum / denom
    gemb = jax.lax.dot_general(wsum, gt, (((1,), (0,)), ((), ())),
                               preferred_element_type=jnp.float32)
    x = jnp.concatenate([ue_ref[...], ie_ref[...], gemb], axis=1)
    z = jax.lax.dot_general(x, w_ref[...], (((1,), (1,)), ((), ())),
                            preferred_element_type=jnp.float32) + b_ref[...]
    z_ref[...] = z

    @pl.when(pl.program_id(0) == 0)
    def _():
        s_ref[...] = jnp.zeros_like(s_ref)
        ss_ref[...] = jnp.zeros_like(ss_ref)

    s_ref[...] += jnp.sum(z, axis=0, keepdims=True)
    ss_ref[...] += jnp.sum(z * z, axis=0, keepdims=True)


def _bn_mm_body(z_ref, s_ref, ss_ref, g_ref, be_ref, w_ref, b_ref,
                zo_ref, so_ref, sso_ref, *, B):
    mean = s_ref[...] / B
    var = ss_ref[...] / B - mean * mean
    inv = lax.rsqrt(var + _EPS)
    h = jnp.maximum((z_ref[...] - mean) * (inv * g_ref[...]) + be_ref[...], 0.0)
    z = jax.lax.dot_general(h, w_ref[...], (((1,), (1,)), ((), ())),
                            preferred_element_type=jnp.float32) + b_ref[...]
    zo_ref[...] = z

    @pl.when(pl.program_id(0) == 0)
    def _():
        so_ref[...] = jnp.zeros_like(so_ref)
        sso_ref[...] = jnp.zeros_like(sso_ref)

    so_ref[...] += jnp.sum(z, axis=0, keepdims=True)
    sso_ref[...] += jnp.sum(z * z, axis=0, keepdims=True)


def _head_body(z_ref, s_ref, ss_ref, g_ref, be_ref, w_ref, b_ref, o_ref, *, B):
    mean = s_ref[...] / B
    var = ss_ref[...] / B - mean * mean
    inv = lax.rsqrt(var + _EPS)
    h = jnp.maximum((z_ref[...] - mean) * (inv * g_ref[...]) + be_ref[...], 0.0)
    o = jnp.sum(h * w_ref[...], axis=1, keepdims=True) + b_ref[0, 0]
    o_ref[...] = jax.nn.sigmoid(o) * 5.0


def kernel(user_ids, item_ids, genre_ids, user_table, item_table, genre_table,
           attn_w, attn_b, W0, b0, gamma0, beta0, W1, b1, gamma1, beta1,
           W2, b2, gamma2, beta2, Wp, bp):
    B = user_ids.shape[0]
    G = genre_ids.shape[1]
    NG, DG = genre_table.shape
    D = user_table.shape[1]
    TB = 2048
    nt = B // TB
    f32 = jnp.float32

    ue, ie = _sc_gather(user_ids.astype(jnp.int32), item_ids.astype(jnp.int32),
                        user_table, item_table)

    NGP = 32
    gt_pad = jnp.zeros((NGP, DG), f32).at[:NG].set(genre_table)
    gid = genre_ids.astype(jnp.int32)

    H0 = W0.shape[0]
    H1 = W1.shape[0]
    H2 = W2.shape[0]

    row = lambda v: v.reshape(1, -1)

    z0, s0, ss0 = pl.pallas_call(
        functools.partial(_p0_body, G=G, NGP=NGP),
        grid=(nt,),
        in_specs=[
            pl.BlockSpec((TB, G), lambda i: (i, 0)),
            pl.BlockSpec((TB, D), lambda i: (i, 0)),
            pl.BlockSpec((TB, D), lambda i: (i, 0)),
            pl.BlockSpec((NGP, DG), lambda i: (0, 0)),
            pl.BlockSpec((1, DG), lambda i: (0, 0)),
            pl.BlockSpec((H0, 2 * D + DG), lambda i: (0, 0)),
            pl.BlockSpec((1, H0), lambda i: (0, 0)),
        ],
        out_specs=[
            pl.BlockSpec((TB, H0), lambda i: (i, 0)),
            pl.BlockSpec((1, H0), lambda i: (0, 0)),
            pl.BlockSpec((1, H0), lambda i: (0, 0)),
        ],
        out_shape=[
            jax.ShapeDtypeStruct((B, H0), f32),
            jax.ShapeDtypeStruct((1, H0), f32),
            jax.ShapeDtypeStruct((1, H0), f32),
        ],
    )(gid, ue, ie, gt_pad, attn_w, W0, row(b0))

    def bn_mm(z, s, ss, gamma, beta, W, b, dp, dn):
        return pl.pallas_call(
            functools.partial(_bn_mm_body, B=B),
            grid=(nt,),
            in_specs=[
                pl.BlockSpec((TB, dp), lambda i: (i, 0)),
                pl.BlockSpec((1, dp), lambda i: (0, 0)),
                pl.BlockSpec((1, dp), lambda i: (0, 0)),
                pl.BlockSpec((1, dp), lambda i: (0, 0)),
                pl.BlockSpec((1, dp), lambda i: (0, 0)),
                pl.BlockSpec((dn, dp), lambda i: (0, 0)),
                pl.BlockSpec((1, dn), lambda i: (0, 0)),
            ],
            out_specs=[
                pl.BlockSpec((TB, dn), lambda i: (i, 0)),
                pl.BlockSpec((1, dn), lambda i: (0, 0)),
                pl.BlockSpec((1, dn), lambda i: (0, 0)),
            ],
            out_shape=[
                jax.ShapeDtypeStruct((B, dn), f32),
                jax.ShapeDtypeStruct((1, dn), f32),
                jax.ShapeDtypeStruct((1, dn), f32),
            ],
        )(z, s, ss, row(gamma), row(beta), W, row(b))

    z1, s1, ss1 = bn_mm(z0, s0, ss0, gamma0, beta0, W1, b1, H0, H1)
    z2, s2, ss2 = bn_mm(z1, s1, ss1, gamma1, beta1, W2, b2, H1, H2)

    out = pl.pallas_call(
        functools.partial(_head_body, B=B),
        grid=(nt,),
        in_specs=[
            pl.BlockSpec((TB, H2), lambda i: (i, 0)),
            pl.BlockSpec((1, H2), lambda i: (0, 0)),
            pl.BlockSpec((1, H2), lambda i: (0, 0)),
            pl.BlockSpec((1, H2), lambda i: (0, 0)),
            pl.BlockSpec((1, H2), lambda i: (0, 0)),
            pl.BlockSpec((1, H2), lambda i: (0, 0)),
            pl.BlockSpec((1, 1), lambda i: (0, 0)),
        ],
        out_specs=[pl.BlockSpec((TB, 1), lambda i: (i, 0))],
        out_shape=[jax.ShapeDtypeStruct((B, 1), f32)],
    )(z2, s2, ss2, row(gamma2), row(beta2), Wp, row(bp))[0]

    return out.reshape(B)


# speedup vs baseline: 6.5316x; 1.1425x over previous
import functools

import jax
import jax.numpy as jnp
from jax import lax
from jax.experimental import pallas as pl
from jax.experimental.pallas import tpu as pltpu
from jax.experimental.pallas import tpu_sc as plsc

_EPS = 1e-5

_NC = 2
_NS = 16
_NW = _NC * _NS
_SUB = 128


def _sc_gather(user_ids, item_ids, user_table, item_table):
    B = user_ids.shape[0]
    D = user_table.shape[1]
    ch = B // _NW
    nchunk = ch // _SUB

    uids2 = user_ids.reshape(B // _SUB, _SUB)
    iids2 = item_ids.reshape(B // _SUB, _SUB)

    mesh = plsc.VectorSubcoreMesh(core_axis_name="c", subcore_axis_name="s")

    @functools.partial(
        pl.kernel,
        mesh=mesh,
        out_type=(
            jax.ShapeDtypeStruct((B, D), jnp.float32),
            jax.ShapeDtypeStruct((B, D), jnp.float32),
        ),
        scratch_types=[
            pltpu.VMEM((nchunk, _SUB), jnp.int32),
            pltpu.VMEM((ch, D), jnp.float32),
            pltpu.SemaphoreType.DMA,
        ],
    )
    def gk(uids, iids, ut, it, ue_out, ie_out, idx_v, rows_v, sem):
        wid = lax.axis_index("s") * _NC + lax.axis_index("c")
        base = wid * ch
        for ids, tbl, out in ((uids, ut, ue_out), (iids, it, ie_out)):
            pltpu.sync_copy(ids.at[pl.ds(wid * nchunk, nchunk)], idx_v)
            handles = []
            for j in range(nchunk):
                handles.append(
                    pltpu.async_copy(
                        tbl.at[idx_v.at[j]],
                        rows_v.at[pl.ds(j * _SUB, _SUB)],
                        sem,
                    )
                )
            for h in handles:
                h.wait()
            pltpu.sync_copy(rows_v, out.at[pl.ds(base, ch)])

    return gk(uids2, iids2, user_table, item_table)


def _mlp_body(gid_ref, ue_ref, ie_ref, gt_ref, aw_ref,
              w0_ref, b0_ref, g0_ref, be0_ref,
              w1_ref, b1_ref, g1_ref, be1_ref,
              w2_ref, b2_ref, g2_ref, be2_ref,
              wp_ref, bp_ref,
              o_ref,
              z0_sc, z1_sc, z2_sc, s0_sc, ss0_sc, s1_sc, ss1_sc, s2_sc, ss2_sc,
              *, B, TB, G, NGP):
    p = pl.program_id(0)
    i = pl.program_id(1)
    row = pl.ds(i * TB, TB)

    @pl.when(p == 0)
    def _pass0():
        gid = gid_ref[...]
        gt = gt_ref[...]
        aw = aw_ref[...]
        lt = jnp.sum(gt * aw, axis=1)
        iota_t = lax.broadcasted_iota(jnp.int32, (TB, NGP), 1)
        ohs, ls = [], []
        for g in range(G):
            oh = (gid[:, g:g + 1] == iota_t).astype(jnp.float32)
            ohs.append(oh)
            ls.append(jnp.sum(oh * lt[None, :], axis=1, keepdims=True))
        m = ls[0]
        for g in range(1, G):
            m = jnp.maximum(m, ls[g])
        es = [jnp.exp(l - m) for l in ls]
        denom = es[0]
        for g in range(1, G):
            denom = denom + es[g]
        wsum = es[0] * ohs[0]
        for g in range(1, G):
            wsum = wsum + es[g] * ohs[g]
        wsum = wsum / denom
        gemb = lax.dot_general(wsum, gt, (((1,), (0,)), ((), ())),
                               preferred_element_type=jnp.float32)
        x = jnp.concatenate([ue_ref[...], ie_ref[...], gemb], axis=1)
        z = lax.dot_general(x, w0_ref[...], (((1,), (1,)), ((), ())),
                            preferred_element_type=jnp.float32) + b0_ref[...]
        z0_sc[row, :] = z

        @pl.when(i == 0)
        def _():
            s0_sc[...] = jnp.zeros_like(s0_sc)
            ss0_sc[...] = jnp.zeros_like(ss0_sc)

        s0_sc[...] += jnp.sum(z, axis=0, keepdims=True)
        ss0_sc[...] += jnp.sum(z * z, axis=0, keepdims=True)

    def bn_relu(z, s_sc, ss_sc, g_ref, be_ref):
        mean = s_sc[...] / B
        var = ss_sc[...] / B - mean * mean
        inv = lax.rsqrt(var + _EPS)
        return jnp.maximum((z - mean) * (inv * g_ref[...]) + be_ref[...], 0.0)

    @pl.when(p == 1)
    def _pass1():
        h = bn_relu(z0_sc[row, :], s0_sc, ss0_sc, g0_ref, be0_ref)
        z = lax.dot_general(h, w1_ref[...], (((1,), (1,)), ((), ())),
                            preferred_element_type=jnp.float32) + b1_ref[...]
        z1_sc[row, :] = z

        @pl.when(i == 0)
        def _():
            s1_sc[...] = jnp.zeros_like(s1_sc)
            ss1_sc[...] = jnp.zeros_like(ss1_sc)

        s1_sc[...] += jnp.sum(z, axis=0, keepdims=True)
        ss1_sc[...] += jnp.sum(z * z, axis=0, keepdims=True)

    @pl.when(p == 2)
    def _pass2():
        h = bn_relu(z1_sc[row, :], s1_sc, ss1_sc, g1_ref, be1_ref)
        z = lax.dot_general(h, w2_ref[...], (((1,), (1,)), ((), ())),
                            preferred_element_type=jnp.float32) + b2_ref[...]
        z2_sc[row, :] = z

        @pl.when(i == 0)
        def _():
            s2_sc[...] = jnp.zeros_like(s2_sc)
            ss2_sc[...] = jnp.zeros_like(ss2_sc)

        s2_sc[...] += jnp.sum(z, axis=0, keepdims=True)
        ss2_sc[...] += jnp.sum(z * z, axis=0, keepdims=True)

    @pl.when(p == 3)
    def _pass3():
        h = bn_relu(z2_sc[row, :], s2_sc, ss2_sc, g2_ref, be2_ref)
        o = jnp.sum(h * wp_ref[...], axis=1, keepdims=True) + bp_ref[0, 0]
        o_ref[...] = jax.nn.sigmoid(o) * 5.0


def kernel(user_ids, item_ids, genre_ids, user_table, item_table, genre_table,
           attn_w, attn_b, W0, b0, gamma0, beta0, W1, b1, gamma1, beta1,
           W2, b2, gamma2, beta2, Wp, bp):
    B = user_ids.shape[0]
    G = genre_ids.shape[1]
    NG, DG = genre_table.shape
    D = user_table.shape[1]
    TB = 2048
    nt = B // TB
    f32 = jnp.float32

    ue, ie = _sc_gather(user_ids.astype(jnp.int32), item_ids.astype(jnp.int32),
                        user_table, item_table)

    NGP = 32
    gt_pad = jnp.zeros((NGP, DG), f32).at[:NG].set(genre_table)
    gid = genre_ids.astype(jnp.int32)

    H0, H1, H2 = W0.shape[0], W1.shape[0], W2.shape[0]
    row = lambda v: v.reshape(1, -1)

    p0 = lambda p, i: (jnp.where(p == 0, i, 0), 0)
    fix = lambda p, i: (0, 0)
    full = lambda s: pl.BlockSpec(s, fix)

    out = pl.pallas_call(
        functools.partial(_mlp_body, B=B, TB=TB, G=G, NGP=NGP),
        grid=(4, nt),
        in_specs=[
            pl.BlockSpec((TB, G), p0),
            pl.BlockSpec((TB, D), p0),
            pl.BlockSpec((TB, D), p0),
            full((NGP, DG)),
            full((1, DG)),
            full((H0, 2 * D + DG)), full((1, H0)), full((1, H0)), full((1, H0)),
            full((H1, H0)), full((1, H1)), full((1, H1)), full((1, H1)),
            full((H2, H1)), full((1, H2)), full((1, H2)), full((1, H2)),
            full((1, H2)), full((1, 1)),
        ],
        out_specs=pl.BlockSpec((TB, 1), lambda p, i: (jnp.where(p == 3, i, 0), 0)),
        out_shape=jax.ShapeDtypeStruct((B, 1), f32),
        scratch_shapes=[
            pltpu.VMEM((B, H0), f32),
            pltpu.VMEM((B, H1), f32),
            pltpu.VMEM((B, H2), f32),
            pltpu.VMEM((1, H0), f32), pltpu.VMEM((1, H0), f32),
            pltpu.VMEM((1, H1), f32), pltpu.VMEM((1, H1), f32),
            pltpu.VMEM((1, H2), f32), pltpu.VMEM((1, H2), f32),
        ],
    )(gid, ue, ie, gt_pad, attn_w,
      W0, row(b0), row(gamma0), row(beta0),
      W1, row(b1), row(gamma1), row(beta1),
      W2, row(b2), row(gamma2), row(beta2),
      Wp, row(bp))

    return out.reshape(B)


# speedup vs baseline: 6.7179x; 1.0285x over previous
import functools

import jax
import jax.numpy as jnp
from jax import lax
from jax.experimental import pallas as pl
from jax.experimental.pallas import tpu as pltpu
from jax.experimental.pallas import tpu_sc as plsc

_EPS = 1e-5

_NC = 2
_NS = 16
_NW = _NC * _NS
_SUB = 128


def _sc_gather(user_ids, item_ids, user_table, item_table):
    B = user_ids.shape[0]
    D = user_table.shape[1]
    ch = B // _NW
    nchunk = ch // _SUB

    uids2 = user_ids.reshape(B // _SUB, _SUB)
    iids2 = item_ids.reshape(B // _SUB, _SUB)

    mesh = plsc.VectorSubcoreMesh(core_axis_name="c", subcore_axis_name="s")

    @functools.partial(
        pl.kernel,
        mesh=mesh,
        out_type=(
            jax.ShapeDtypeStruct((B, D), jnp.float32),
            jax.ShapeDtypeStruct((B, D), jnp.float32),
        ),
        scratch_types=[
            pltpu.VMEM((nchunk, _SUB), jnp.int32),
            pltpu.VMEM((ch, D), jnp.float32),
            pltpu.SemaphoreType.DMA,
        ],
    )
    def gk(uids, iids, ut, it, ue_out, ie_out, idx_v, rows_v, sem):
        wid = lax.axis_index("s") * _NC + lax.axis_index("c")
        base = wid * ch
        for ids, tbl, out in ((uids, ut, ue_out), (iids, it, ie_out)):
            pltpu.sync_copy(ids.at[pl.ds(wid * nchunk, nchunk)], idx_v)
            handles = []
            for j in range(nchunk):
                handles.append(
                    pltpu.async_copy(
                        tbl.at[idx_v.at[j]],
                        rows_v.at[pl.ds(j * _SUB, _SUB)],
                        sem,
                    )
                )
            for h in handles:
                h.wait()
            pltpu.sync_copy(rows_v, out.at[pl.ds(base, ch)])

    return gk(uids2, iids2, user_table, item_table)


def _genre_body(gid_ref, gt_ref, aw_ref, ge_ref, *, G, NGP):
    gid = gid_ref[...]
    gt = gt_ref[...]
    aw = aw_ref[...]
    lt = jnp.sum(gt * aw, axis=1)
    tb = gid.shape[0]
    iota_t = lax.broadcasted_iota(jnp.int32, (tb, NGP), 1)
    cmps, ls = [], []
    for g in range(G):
        cmp = gid[:, g:g + 1] == iota_t
        cmps.append(cmp)
        ls.append(jnp.max(jnp.where(cmp, lt[None, :], -jnp.inf),
                          axis=1, keepdims=True))
    m = ls[0]
    for g in range(1, G):
        m = jnp.maximum(m, ls[g])
    es = [jnp.exp(l - m) for l in ls]
    denom = es[0]
    for g in range(1, G):
        denom = denom + es[g]
    wsum = jnp.where(cmps[0], es[0], 0.0)
    for g in range(1, G):
        wsum = wsum + jnp.where(cmps[g], es[g], 0.0)
    wsum = wsum / denom
    ge_ref[...] = lax.dot_general(wsum, gt, (((1,), (0,)), ((), ())),
                                  preferred_element_type=jnp.float32)


def _mlp_body(ue_ref, ie_ref, gemb_ref,
              w0_ref, b0_ref, g0_ref, be0_ref,
              w1_ref, b1_ref, g1_ref, be1_ref,
              w2_ref, b2_ref, g2_ref, be2_ref,
              wp_ref, bp_ref,
              o_ref,
              z0_sc, z1_sc, z2_sc, s0_sc, ss0_sc, s1_sc, ss1_sc, s2_sc, ss2_sc,
              *, B, TB):
    p = pl.program_id(0)
    i = pl.program_id(1)
    row = pl.ds(i * TB, TB)

    @pl.when(p == 0)
    def _pass0():
        x = jnp.concatenate([ue_ref[...], ie_ref[...], gemb_ref[...]], axis=1)
        z = lax.dot_general(x, w0_ref[...], (((1,), (1,)), ((), ())),
                            preferred_element_type=jnp.float32) + b0_ref[...]
        z0_sc[row, :] = z

        @pl.when(i == 0)
        def _():
            s0_sc[...] = jnp.zeros_like(s0_sc)
            ss0_sc[...] = jnp.zeros_like(ss0_sc)

        s0_sc[...] += jnp.sum(z, axis=0, keepdims=True)
        ss0_sc[...] += jnp.sum(z * z, axis=0, keepdims=True)

    def bn_relu(z, s_sc, ss_sc, g_ref, be_ref):
        mean = s_sc[...] / B
        var = ss_sc[...] / B - mean * mean
        a = lax.rsqrt(var + _EPS) * g_ref[...]
        c = be_ref[...] - mean * a
        return jnp.maximum(z * a + c, 0.0)

    @pl.when(p == 1)
    def _pass1():
        h = bn_relu(z0_sc[row, :], s0_sc, ss0_sc, g0_ref, be0_ref)
        z = lax.dot_general(h, w1_ref[...], (((1,), (1,)), ((), ())),
                            preferred_element_type=jnp.float32) + b1_ref[...]
        z1_sc[row, :] = z

        @pl.when(i == 0)
        def _():
            s1_sc[...] = jnp.zeros_like(s1_sc)
            ss1_sc[...] = jnp.zeros_like(ss1_sc)

        s1_sc[...] += jnp.sum(z, axis=0, keepdims=True)
        ss1_sc[...] += jnp.sum(z * z, axis=0, keepdims=True)

    @pl.when(p == 2)
    def _pass2():
        h = bn_relu(z1_sc[row, :], s1_sc, ss1_sc, g1_ref, be1_ref)
        z = lax.dot_general(h, w2_ref[...], (((1,), (1,)), ((), ())),
                            preferred_element_type=jnp.float32) + b2_ref[...]
        z2_sc[row, :] = z

        @pl.when(i == 0)
        def _():
            s2_sc[...] = jnp.zeros_like(s2_sc)
            ss2_sc[...] = jnp.zeros_like(ss2_sc)

        s2_sc[...] += jnp.sum(z, axis=0, keepdims=True)
        ss2_sc[...] += jnp.sum(z * z, axis=0, keepdims=True)

    @pl.when(p == 3)
    def _pass3():
        h = bn_relu(z2_sc[row, :], s2_sc, ss2_sc, g2_ref, be2_ref)
        o = jnp.sum(h * wp_ref[...], axis=1, keepdims=True) + bp_ref[0, 0]
        o_ref[...] = jax.nn.sigmoid(o) * 5.0


def kernel(user_ids, item_ids, genre_ids, user_table, item_table, genre_table,
           attn_w, attn_b, W0, b0, gamma0, beta0, W1, b1, gamma1, beta1,
           W2, b2, gamma2, beta2, Wp, bp):
    B = user_ids.shape[0]
    G = genre_ids.shape[1]
    NG, DG = genre_table.shape
    D = user_table.shape[1]
    TB = 2048
    nt = B // TB
    f32 = jnp.float32

    ue, ie = _sc_gather(user_ids.astype(jnp.int32), item_ids.astype(jnp.int32),
                        user_table, item_table)

    NGP = 32
    gt_pad = jnp.zeros((NGP, DG), f32).at[:NG].set(genre_table)
    gid = genre_ids.astype(jnp.int32)

    gemb = pl.pallas_call(
        functools.partial(_genre_body, G=G, NGP=NGP),
        grid=(nt,),
        in_specs=[
            pl.BlockSpec((TB, G), lambda i: (i, 0)),
            pl.BlockSpec((NGP, DG), lambda i: (0, 0)),
            pl.BlockSpec((1, DG), lambda i: (0, 0)),
        ],
        out_specs=pl.BlockSpec((TB, DG), lambda i: (i, 0)),
        out_shape=jax.ShapeDtypeStruct((B, DG), f32),
    )(gid, gt_pad, attn_w)

    H0, H1, H2 = W0.shape[0], W1.shape[0], W2.shape[0]
    row = lambda v: v.reshape(1, -1)

    p0 = lambda p, i: (jnp.where(p == 0, i, 0), 0)
    fix = lambda p, i: (0, 0)
    full = lambda s: pl.BlockSpec(s, fix)

    out = pl.pallas_call(
        functools.partial(_mlp_body, B=B, TB=TB),
        grid=(4, nt),
        in_specs=[
            pl.BlockSpec((TB, D), p0),
            pl.BlockSpec((TB, D), p0),
            pl.BlockSpec((TB, DG), p0),
            full((H0, 2 * D + DG)), full((1, H0)), full((1, H0)), full((1, H0)),
            full((H1, H0)), full((1, H1)), full((1, H1)), full((1, H1)),
            full((H2, H1)), full((1, H2)), full((1, H2)), full((1, H2)),
            full((1, H2)), full((1, 1)),
        ],
        out_specs=pl.BlockSpec((TB, 1), lambda p, i: (jnp.where(p == 3, i, 0), 0)),
        out_shape=jax.ShapeDtypeStruct((B, 1), f32),
        scratch_shapes=[
            pltpu.VMEM((B, H0), f32),
            pltpu.VMEM((B, H1), f32),
            pltpu.VMEM((B, H2), f32),
            pltpu.VMEM((1, H0), f32), pltpu.VMEM((1, H0), f32),
            pltpu.VMEM((1, H1), f32), pltpu.VMEM((1, H1), f32),
            pltpu.VMEM((1, H2), f32), pltpu.VMEM((1, H2), f32),
        ],
    )(ue, ie, gemb,
      W0, row(b0), row(gamma0), row(beta0),
      W1, row(b1), row(gamma1), row(beta1),
      W2, row(b2), row(gamma2), row(beta2),
      Wp, row(bp))

    return out.reshape(B)


# speedup vs baseline: 7.5122x; 1.1182x over previous
import functools

import jax
import jax.numpy as jnp
from jax import lax
from jax.experimental import pallas as pl
from jax.experimental.pallas import tpu as pltpu
from jax.experimental.pallas import tpu_sc as plsc

_EPS = 1e-5

_NC = 2
_NS = 16
_NW = _NC * _NS
_SUB = 128


def _sc_gather(user_ids, item_ids, user_table, item_table):
    B = user_ids.shape[0]
    D = user_table.shape[1]
    ch = B // _NW
    nchunk = ch // _SUB

    uids2 = user_ids.reshape(B // _SUB, _SUB)
    iids2 = item_ids.reshape(B // _SUB, _SUB)

    mesh = plsc.VectorSubcoreMesh(core_axis_name="c", subcore_axis_name="s")

    @functools.partial(
        pl.kernel,
        mesh=mesh,
        out_type=(
            jax.ShapeDtypeStruct((B, D), jnp.float32),
            jax.ShapeDtypeStruct((B, D), jnp.float32),
        ),
        scratch_types=[
            pltpu.VMEM((nchunk, _SUB), jnp.int32),
            pltpu.VMEM((ch, D), jnp.float32),
            pltpu.SemaphoreType.DMA,
        ],
    )
    def gk(uids, iids, ut, it, ue_out, ie_out, idx_v, rows_v, sem):
        wid = lax.axis_index("s") * _NC + lax.axis_index("c")
        base = wid * ch
        for ids, tbl, out in ((uids, ut, ue_out), (iids, it, ie_out)):
            pltpu.sync_copy(ids.at[pl.ds(wid * nchunk, nchunk)], idx_v)
            handles = []
            for j in range(nchunk):
                handles.append(
                    pltpu.async_copy(
                        tbl.at[idx_v.at[j]],
                        rows_v.at[pl.ds(j * _SUB, _SUB)],
                        sem,
                    )
                )
            for h in handles:
                h.wait()
            pltpu.sync_copy(rows_v, out.at[pl.ds(base, ch)])

    return gk(uids2, iids2, user_table, item_table)


def _genre_body(gid_ref, gt_ref, aw_ref, ge_ref, *, G, NGP):
    gid = gid_ref[...]
    gt = gt_ref[...]
    aw = aw_ref[...]
    lt = jnp.sum(gt * aw, axis=1)
    tb = gid.shape[0]
    iota_t = lax.broadcasted_iota(jnp.int32, (tb, NGP), 1)
    cmps, ls = [], []
    for g in range(G):
        cmp = gid[:, g:g + 1] == iota_t
        cmps.append(cmp)
        ls.append(jnp.max(jnp.where(cmp, lt[None, :], -jnp.inf),
                          axis=1, keepdims=True))
    m = ls[0]
    for g in range(1, G):
        m = jnp.maximum(m, ls[g])
    es = [jnp.exp(l - m) for l in ls]
    denom = es[0]
    for g in range(1, G):
        denom = denom + es[g]
    wsum = jnp.where(cmps[0], es[0], 0.0)
    for g in range(1, G):
        wsum = wsum + jnp.where(cmps[g], es[g], 0.0)
    wsum = wsum / denom
    ge_ref[...] = lax.dot_general(wsum, gt, (((1,), (0,)), ((), ())),
                                  preferred_element_type=jnp.float32)


def _mlp_body(ue_ref, ie_ref, gemb_ref,
              w0u_ref, w0i_ref, w0g_ref, b0_ref, g0_ref, be0_ref,
              w1_ref, b1_ref, g1_ref, be1_ref,
              w2_ref, b2_ref, g2_ref, be2_ref,
              wp_ref, bp_ref,
              o_ref,
              z0_sc, z1_sc, z2_sc, s0_sc, ss0_sc, s1_sc, ss1_sc, s2_sc, ss2_sc,
              *, B, TB):
    p = pl.program_id(0)
    i = pl.program_id(1)
    row = pl.ds(pl.multiple_of(i * TB, TB), TB)
    bf16 = jnp.bfloat16

    def colstats(z, s_sc, ss_sc):
        @pl.when(i == 0)
        def _():
            s_sc[...] = jnp.zeros_like(s_sc)
            ss_sc[...] = jnp.zeros_like(ss_sc)

        s_sc[...] += jnp.sum(z, axis=0, keepdims=True)
        ss_sc[...] += jnp.sum(z * z, axis=0, keepdims=True)

    def mm(a, w_ref):
        return lax.dot_general(a.astype(bf16), w_ref[...].astype(bf16),
                               (((1,), (1,)), ((), ())),
                               preferred_element_type=jnp.float32)

    @pl.when(p == 0)
    def _pass0():
        z = mm(ue_ref[...], w0u_ref) + mm(ie_ref[...], w0i_ref)
        z = z + mm(gemb_ref[...], w0g_ref) + b0_ref[...]
        z0_sc[row, :] = z
        colstats(z, s0_sc, ss0_sc)

    def bn_relu(z, s_sc, ss_sc, g_ref, be_ref):
        mean = s_sc[...] / B
        var = ss_sc[...] / B - mean * mean
        a = lax.rsqrt(var + _EPS) * g_ref[...]
        c = be_ref[...] - mean * a
        return jnp.maximum(z * a + c, 0.0)

    @pl.when(p == 1)
    def _pass1():
        h = bn_relu(z0_sc[row, :], s0_sc, ss0_sc, g0_ref, be0_ref)
        z = mm(h, w1_ref) + b1_ref[...]
        z1_sc[row, :] = z
        colstats(z, s1_sc, ss1_sc)

    @pl.when(p == 2)
    def _pass2():
        h = bn_relu(z1_sc[row, :], s1_sc, ss1_sc, g1_ref, be1_ref)
        z = mm(h, w2_ref) + b2_ref[...]
        z2_sc[row, :] = z
        colstats(z, s2_sc, ss2_sc)

    @pl.when(p == 3)
    def _pass3():
        h = bn_relu(z2_sc[row, :], s2_sc, ss2_sc, g2_ref, be2_ref)
        ot = lax.dot_general(wp_ref[...].astype(bf16), h.astype(bf16),
                             (((1,), (1,)), ((), ())),
                             preferred_element_type=jnp.float32) + bp_ref[0, 0]
        o_ref[...] = (jax.nn.sigmoid(ot) * 5.0).reshape(1, 1, TB)


def kernel(user_ids, item_ids, genre_ids, user_table, item_table, genre_table,
           attn_w, attn_b, W0, b0, gamma0, beta0, W1, b1, gamma1, beta1,
           W2, b2, gamma2, beta2, Wp, bp):
    B = user_ids.shape[0]
    G = genre_ids.shape[1]
    NG, DG = genre_table.shape
    D = user_table.shape[1]
    TB = 4096
    nt = B // TB
    f32 = jnp.float32

    ue, ie = _sc_gather(user_ids.astype(jnp.int32), item_ids.astype(jnp.int32),
                        user_table, item_table)

    NGP = 32
    gt_pad = jnp.zeros((NGP, DG), f32).at[:NG].set(genre_table)
    gid = genre_ids.astype(jnp.int32)

    gemb = pl.pallas_call(
        functools.partial(_genre_body, G=G, NGP=NGP),
        grid=(nt,),
        in_specs=[
            pl.BlockSpec((TB, G), lambda i: (i, 0)),
            pl.BlockSpec((NGP, DG), lambda i: (0, 0)),
            pl.BlockSpec((1, DG), lambda i: (0, 0)),
        ],
        out_specs=pl.BlockSpec((TB, DG), lambda i: (i, 0)),
        out_shape=jax.ShapeDtypeStruct((B, DG), f32),
    )(gid, gt_pad, attn_w)

    H0, H1, H2 = W0.shape[0], W1.shape[0], W2.shape[0]
    row = lambda v: v.reshape(1, -1)

    p0 = lambda p, i: (jnp.where(p == 0, i, 0), 0)
    fix = lambda p, i: (0, 0)
    full = lambda s: pl.BlockSpec(s, fix)

    out = pl.pallas_call(
        functools.partial(_mlp_body, B=B, TB=TB),
        grid=(4, nt),
        in_specs=[
            pl.BlockSpec((TB, D), p0),
            pl.BlockSpec((TB, D), p0),
            pl.BlockSpec((TB, DG), p0),
            full((H0, D)), full((H0, D)), full((H0, DG)),
            full((1, H0)), full((1, H0)), full((1, H0)),
            full((H1, H0)), full((1, H1)), full((1, H1)), full((1, H1)),
            full((H2, H1)), full((1, H2)), full((1, H2)), full((1, H2)),
            full((1, H2)), full((1, 1)),
        ],
        out_specs=pl.BlockSpec((1, 1, TB),
                               lambda p, i: (jnp.where(p == 3, i, 0), 0, 0)),
        out_shape=jax.ShapeDtypeStruct((nt, 1, TB), f32),
        scratch_shapes=[
            pltpu.VMEM((B, H0), f32),
            pltpu.VMEM((B, H1), f32),
            pltpu.VMEM((B, H2), f32),
            pltpu.VMEM((1, H0), f32), pltpu.VMEM((1, H0), f32),
            pltpu.VMEM((1, H1), f32), pltpu.VMEM((1, H1), f32),
            pltpu.VMEM((1, H2), f32), pltpu.VMEM((1, H2), f32),
        ],
    )(ue, ie, gemb,
      W0[:, :D], W0[:, D:2 * D], W0[:, 2 * D:],
      row(b0), row(gamma0), row(beta0),
      W1, row(b1), row(gamma1), row(beta1),
      W2, row(b2), row(gamma2), row(beta2),
      Wp, row(bp))

    return out.reshape(B)


# speedup vs baseline: 9.7988x; 1.3044x over previous
import functools

import jax
import jax.numpy as jnp
from jax import lax
from jax.experimental import pallas as pl
from jax.experimental.pallas import tpu as pltpu
from jax.experimental.pallas import tpu_sc as plsc

_EPS = 1e-5

_NC = 2
_NS = 16
_NW = _NC * _NS
_SUB = 128


def _sc_gather(user_ids, item_ids, user_table, item_table):
    B = user_ids.shape[0]
    D = user_table.shape[1]
    ch = B // _NW
    nchunk = ch // _SUB

    uids2 = user_ids.reshape(B // _SUB, _SUB)
    iids2 = item_ids.reshape(B // _SUB, _SUB)

    mesh = plsc.VectorSubcoreMesh(core_axis_name="c", subcore_axis_name="s")

    @functools.partial(
        pl.kernel,
        mesh=mesh,
        out_type=(
            jax.ShapeDtypeStruct((B, D), jnp.float32),
            jax.ShapeDtypeStruct((B, D), jnp.float32),
        ),
        scratch_types=[
            pltpu.VMEM((nchunk, _SUB), jnp.int32),
            pltpu.VMEM((ch, D), jnp.float32),
            pltpu.SemaphoreType.DMA,
        ],
    )
    def gk(uids, iids, ut, it, ue_out, ie_out, idx_v, rows_v, sem):
        wid = lax.axis_index("s") * _NC + lax.axis_index("c")
        base = wid * ch
        for ids, tbl, out in ((uids, ut, ue_out), (iids, it, ie_out)):
            pltpu.sync_copy(ids.at[pl.ds(wid * nchunk, nchunk)], idx_v)
            handles = []
            for j in range(nchunk):
                handles.append(
                    pltpu.async_copy(
                        tbl.at[idx_v.at[j]],
                        rows_v.at[pl.ds(j * _SUB, _SUB)],
                        sem,
                    )
                )
            for h in handles:
                h.wait()
            pltpu.sync_copy(rows_v, out.at[pl.ds(base, ch)])

    return gk(uids2, iids2, user_table, item_table)


def _genre_body(gid_ref, gt_ref, aw_ref, ge_ref, *, G, NGP, TB):
    f32 = jnp.float32
    bf16 = jnp.bfloat16
    L = G * NGP
    gt = gt_ref[...]
    aw = aw_ref[...]
    lt_row = lax.dot_general(aw, gt, (((1,), (1,)), ((), ())),
                             preferred_element_type=f32)
    lt_tiled = jnp.concatenate([lt_row] * G, axis=1)
    rg = lax.broadcasted_iota(jnp.int32, (G, L), 0)
    rl = lax.broadcasted_iota(jnp.int32, (G, L), 1)
    R = (rg == rl // NGP).astype(bf16)
    gid_rep = lax.dot_general(gid_ref[...].astype(bf16), R,
                              (((1,), (0,)), ((), ())),
                              preferred_element_type=f32)
    lane_t = (lax.broadcasted_iota(jnp.int32, (TB, L), 1) % NGP).astype(f32)
    cmp = gid_rep == lane_t
    mlt = jnp.where(cmp, lt_tiled, 0.0)
    l5 = lax.dot_general(mlt.astype(bf16), R, (((1,), (1,)), ((), ())),
                         preferred_element_type=f32)
    m = jnp.max(l5, axis=1, keepdims=True)
    e5 = jnp.exp(l5 - m)
    w5 = e5 / jnp.sum(e5, axis=1, keepdims=True)
    w_rep = lax.dot_general(w5.astype(bf16), R, (((1,), (0,)), ((), ())),
                            preferred_element_type=f32)
    woh = jnp.where(cmp, w_rep, 0.0)
    gt5 = jnp.concatenate([gt] * G, axis=0).astype(bf16)
    ge_ref[...] = lax.dot_general(woh.astype(bf16), gt5,
                                  (((1,), (0,)), ((), ())),
                                  preferred_element_type=f32)


def _mlp_body(ue_ref, ie_ref, gemb_ref,
              w0u_ref, w0i_ref, w0g_ref, b0_ref, g0_ref, be0_ref,
              w1_ref, b1_ref, g1_ref, be1_ref,
              w2_ref, b2_ref, g2_ref, be2_ref,
              wp_ref, bp_ref,
              o_ref,
              z0_sc, z1_sc, z2_sc, s0_sc, ss0_sc, s1_sc, ss1_sc, s2_sc, ss2_sc,
              *, B, TB):
    p = pl.program_id(0)
    i = pl.program_id(1)
    row = pl.ds(pl.multiple_of(i * TB, TB), TB)
    bf16 = jnp.bfloat16

    def colstats(z, s_sc, ss_sc):
        @pl.when(i == 0)
        def _():
            s_sc[...] = jnp.zeros_like(s_sc)
            ss_sc[...] = jnp.zeros_like(ss_sc)

        s_sc[...] += jnp.sum(z, axis=0, keepdims=True)
        ss_sc[...] += jnp.sum(z * z, axis=0, keepdims=True)

    def mm(a, w_ref):
        return lax.dot_general(a.astype(bf16), w_ref[...].astype(bf16),
                               (((1,), (1,)), ((), ())),
                               preferred_element_type=jnp.float32)

    @pl.when(p == 0)
    def _pass0():
        z = mm(ue_ref[...], w0u_ref) + mm(ie_ref[...], w0i_ref)
        z = z + mm(gemb_ref[...], w0g_ref) + b0_ref[...]
        z0_sc[row, :] = z
        colstats(z, s0_sc, ss0_sc)

    def bn_relu(z, s_sc, ss_sc, g_ref, be_ref):
        mean = s_sc[...] / B
        var = ss_sc[...] / B - mean * mean
        a = lax.rsqrt(var + _EPS) * g_ref[...]
        c = be_ref[...] - mean * a
        return jnp.maximum(z * a + c, 0.0)

    @pl.when(p == 1)
    def _pass1():
        h = bn_relu(z0_sc[row, :], s0_sc, ss0_sc, g0_ref, be0_ref)
        z = mm(h, w1_ref) + b1_ref[...]
        z1_sc[row, :] = z
        colstats(z, s1_sc, ss1_sc)

    @pl.when(p == 2)
    def _pass2():
        h = bn_relu(z1_sc[row, :], s1_sc, ss1_sc, g1_ref, be1_ref)
        z = mm(h, w2_ref) + b2_ref[...]
        z2_sc[row, :] = z
        colstats(z, s2_sc, ss2_sc)

    @pl.when(p == 3)
    def _pass3():
        h = bn_relu(z2_sc[row, :], s2_sc, ss2_sc, g2_ref, be2_ref)
        ot = lax.dot_general(wp_ref[...].astype(bf16), h.astype(bf16),
                             (((1,), (1,)), ((), ())),
                             preferred_element_type=jnp.float32) + bp_ref[0, 0]
        o_ref[...] = (jax.nn.sigmoid(ot) * 5.0).reshape(1, 1, TB)


def kernel(user_ids, item_ids, genre_ids, user_table, item_table, genre_table,
           attn_w, attn_b, W0, b0, gamma0, beta0, W1, b1, gamma1, beta1,
           W2, b2, gamma2, beta2, Wp, bp):
    B = user_ids.shape[0]
    G = genre_ids.shape[1]
    NG, DG = genre_table.shape
    D = user_table.shape[1]
    TB = 4096
    nt = B // TB
    f32 = jnp.float32

    ue, ie = _sc_gather(user_ids.astype(jnp.int32), item_ids.astype(jnp.int32),
                        user_table, item_table)

    NGP = 32
    gt_pad = jnp.zeros((NGP, DG), f32).at[:NG].set(genre_table)
    gid = genre_ids.astype(jnp.int32)

    gemb = pl.pallas_call(
        functools.partial(_genre_body, G=G, NGP=NGP, TB=TB),
        grid=(nt,),
        in_specs=[
            pl.BlockSpec((TB, G), lambda i: (i, 0)),
            pl.BlockSpec((NGP, DG), lambda i: (0, 0)),
            pl.BlockSpec((1, DG), lambda i: (0, 0)),
        ],
        out_specs=pl.BlockSpec((TB, DG), lambda i: (i, 0)),
        out_shape=jax.ShapeDtypeStruct((B, DG), f32),
    )(gid, gt_pad, attn_w)

    H0, H1, H2 = W0.shape[0], W1.shape[0], W2.shape[0]
    row = lambda v: v.reshape(1, -1)

    p0 = lambda p, i: (jnp.where(p == 0, i, 0), 0)
    fix = lambda p, i: (0, 0)
    full = lambda s: pl.BlockSpec(s, fix)

    out = pl.pallas_call(
        functools.partial(_mlp_body, B=B, TB=TB),
        grid=(4, nt),
        in_specs=[
            pl.BlockSpec((TB, D), p0),
            pl.BlockSpec((TB, D), p0),
            pl.BlockSpec((TB, DG), p0),
            full((H0, D)), full((H0, D)), full((H0, DG)),
            full((1, H0)), full((1, H0)), full((1, H0)),
            full((H1, H0)), full((1, H1)), full((1, H1)), full((1, H1)),
            full((H2, H1)), full((1, H2)), full((1, H2)), full((1, H2)),
            full((1, H2)), full((1, 1)),
        ],
        out_specs=pl.BlockSpec((1, 1, TB),
                               lambda p, i: (jnp.where(p == 3, i, 0), 0, 0)),
        out_shape=jax.ShapeDtypeStruct((nt, 1, TB), f32),
        scratch_shapes=[
            pltpu.VMEM((B, H0), f32),
            pltpu.VMEM((B, H1), f32),
            pltpu.VMEM((B, H2), f32),
            pltpu.VMEM((1, H0), f32), pltpu.VMEM((1, H0), f32),
            pltpu.VMEM((1, H1), f32), pltpu.VMEM((1, H1), f32),
            pltpu.VMEM((1, H2), f32), pltpu.VMEM((1, H2), f32),
        ],
    )(ue, ie, gemb,
      W0[:, :D], W0[:, D:2 * D], W0[:, 2 * D:],
      row(b0), row(gamma0), row(beta0),
      W1, row(b1), row(gamma1), row(beta1),
      W2, row(b2), row(gamma2), row(beta2),
      Wp, row(bp))

    return out.reshape(B)


# speedup vs baseline: 10.1903x; 1.0400x over previous
import functools

import jax
import jax.numpy as jnp
from jax import lax
from jax.experimental import pallas as pl
from jax.experimental.pallas import tpu as pltpu
from jax.experimental.pallas import tpu_sc as plsc

_EPS = 1e-5

_NC = 2
_NS = 16
_NW = _NC * _NS
_SUB = 128


def _sc_gather(user_ids, item_ids, user_table, item_table):
    B = user_ids.shape[0]
    D = user_table.shape[1]
    ch = B // _NW
    nchunk = ch // _SUB

    uids2 = user_ids.reshape(B // _SUB, _SUB)
    iids2 = item_ids.reshape(B // _SUB, _SUB)

    mesh = plsc.VectorSubcoreMesh(core_axis_name="c", subcore_axis_name="s")

    @functools.partial(
        pl.kernel,
        mesh=mesh,
        out_type=(
            jax.ShapeDtypeStruct((B, D), jnp.float32),
            jax.ShapeDtypeStruct((B, D), jnp.float32),
        ),
        scratch_types=[
            pltpu.VMEM((nchunk, _SUB), jnp.int32),
            pltpu.VMEM((ch, D), jnp.float32),
            pltpu.SemaphoreType.DMA,
        ],
    )
    def gk(uids, iids, ut, it, ue_out, ie_out, idx_v, rows_v, sem):
        wid = lax.axis_index("s") * _NC + lax.axis_index("c")
        base = wid * ch
        for ids, tbl, out in ((uids, ut, ue_out), (iids, it, ie_out)):
            pltpu.sync_copy(ids.at[pl.ds(wid * nchunk, nchunk)], idx_v)
            handles = []
            for j in range(nchunk):
                handles.append(
                    pltpu.async_copy(
                        tbl.at[idx_v.at[j]],
                        rows_v.at[pl.ds(j * _SUB, _SUB)],
                        sem,
                    )
                )
            for h in handles:
                h.wait()
            pltpu.sync_copy(rows_v, out.at[pl.ds(base, ch)])

    return gk(uids2, iids2, user_table, item_table)


def _genre_body(gid_ref, gt_ref, aw_ref, ge_ref, *, G, NGP, TB):
    f32 = jnp.float32
    bf16 = jnp.bfloat16
    L = G * NGP
    gt = gt_ref[...]
    aw = aw_ref[...]
    lt_row = lax.dot_general(aw, gt, (((1,), (1,)), ((), ())),
                             preferred_element_type=f32)
    lt_tiled = jnp.concatenate([lt_row] * G, axis=1)
    rg = lax.broadcasted_iota(jnp.int32, (G, L), 0)
    rl = lax.broadcasted_iota(jnp.int32, (G, L), 1)
    R = (rg == rl // NGP).astype(bf16)
    gid_rep = lax.dot_general(gid_ref[...].astype(bf16), R,
                              (((1,), (0,)), ((), ())),
                              preferred_element_type=f32)
    lane_t = (lax.broadcasted_iota(jnp.int32, (TB, L), 1) % NGP).astype(f32)
    cmp = gid_rep == lane_t
    mlt = jnp.where(cmp, lt_tiled, 0.0)
    l5 = lax.dot_general(mlt.astype(bf16), R, (((1,), (1,)), ((), ())),
                         preferred_element_type=f32)
    m = jnp.max(l5, axis=1, keepdims=True)
    e5 = jnp.exp(l5 - m)
    w5 = e5 / jnp.sum(e5, axis=1, keepdims=True)
    w_rep = lax.dot_general(w5.astype(bf16), R, (((1,), (0,)), ((), ())),
                            preferred_element_type=f32)
    woh = jnp.where(cmp, w_rep, 0.0)
    gt5 = jnp.concatenate([gt] * G, axis=0).astype(bf16)
    ge_ref[...] = lax.dot_general(woh.astype(bf16), gt5,
                                  (((1,), (0,)), ((), ())),
                                  preferred_element_type=f32)


def _mlp_body(ue_ref, ie_ref, gemb_ref,
              w0u_ref, w0i_ref, w0g_ref, b0_ref, g0_ref, be0_ref,
              w1_ref, b1_ref, g1_ref, be1_ref,
              w2_ref, b2_ref, g2_ref, be2_ref,
              wp_ref, bp_ref,
              o_ref,
              z0_sc, s0_sc, ss0_sc,
              *, B, TB, NT):
    s = pl.program_id(0)
    bf16 = jnp.bfloat16

    def mm(a, w_ref):
        return lax.dot_general(a.astype(bf16), w_ref[...].astype(bf16),
                               (((1,), (1,)), ((), ())),
                               preferred_element_type=jnp.float32)

    @pl.when(s < NT)
    def _pass0():
        row = pl.ds(pl.multiple_of(s * TB, TB), TB)
        z = mm(ue_ref[...], w0u_ref) + mm(ie_ref[...], w0i_ref)
        z = z + mm(gemb_ref[...], w0g_ref) + b0_ref[...]
        z0_sc[row, :] = z

        @pl.when(s == 0)
        def _():
            s0_sc[...] = jnp.zeros_like(s0_sc)
            ss0_sc[...] = jnp.zeros_like(ss0_sc)

        s0_sc[...] += jnp.sum(z, axis=0, keepdims=True)
        ss0_sc[...] += jnp.sum(z * z, axis=0, keepdims=True)

    def bn_relu(z, s_row, ss_row, g_ref, be_ref):
        mean = s_row / B
        var = ss_row / B - mean * mean
        a = lax.rsqrt(var + _EPS) * g_ref[...]
        c = be_ref[...] - mean * a
        return jnp.maximum(z * a + c, 0.0)

    @pl.when(s == NT)
    def _rest():
        h0 = bn_relu(z0_sc[...], s0_sc[...], ss0_sc[...], g0_ref, be0_ref)
        z1 = mm(h0, w1_ref) + b1_ref[...]
        h1 = bn_relu(z1, jnp.sum(z1, axis=0, keepdims=True),
                     jnp.sum(z1 * z1, axis=0, keepdims=True),
                     g1_ref, be1_ref)
        z2 = mm(h1, w2_ref) + b2_ref[...]
        h2 = bn_relu(z2, jnp.sum(z2, axis=0, keepdims=True),
                     jnp.sum(z2 * z2, axis=0, keepdims=True),
                     g2_ref, be2_ref)
        ot = lax.dot_general(wp_ref[...].astype(bf16), h2.astype(bf16),
                             (((1,), (1,)), ((), ())),
                             preferred_element_type=jnp.float32) + bp_ref[0, 0]
        o_ref[...] = (jax.nn.sigmoid(ot) * 5.0).reshape(1, 1, B)


def kernel(user_ids, item_ids, genre_ids, user_table, item_table, genre_table,
           attn_w, attn_b, W0, b0, gamma0, beta0, W1, b1, gamma1, beta1,
           W2, b2, gamma2, beta2, Wp, bp):
    B = user_ids.shape[0]
    G = genre_ids.shape[1]
    NG, DG = genre_table.shape
    D = user_table.shape[1]
    TB = 4096
    nt = B // TB
    f32 = jnp.float32

    ue, ie = _sc_gather(user_ids.astype(jnp.int32), item_ids.astype(jnp.int32),
                        user_table, item_table)

    NGP = 32
    gt_pad = jnp.zeros((NGP, DG), f32).at[:NG].set(genre_table)
    gid = genre_ids.astype(jnp.int32)

    gemb = pl.pallas_call(
        functools.partial(_genre_body, G=G, NGP=NGP, TB=TB),
        grid=(nt,),
        in_specs=[
            pl.BlockSpec((TB, G), lambda i: (i, 0)),
            pl.BlockSpec((NGP, DG), lambda i: (0, 0)),
            pl.BlockSpec((1, DG), lambda i: (0, 0)),
        ],
        out_specs=pl.BlockSpec((TB, DG), lambda i: (i, 0)),
        out_shape=jax.ShapeDtypeStruct((B, DG), f32),
    )(gid, gt_pad, attn_w)

    H0, H1, H2 = W0.shape[0], W1.shape[0], W2.shape[0]
    row = lambda v: v.reshape(1, -1)

    p0 = lambda s: (jnp.minimum(s, nt - 1), 0)
    fix = lambda s: (0, 0)
    full = lambda sh: pl.BlockSpec(sh, fix)

    out = pl.pallas_call(
        functools.partial(_mlp_body, B=B, TB=TB, NT=nt),
        grid=(nt + 1,),
        in_specs=[
            pl.BlockSpec((TB, D), p0),
            pl.BlockSpec((TB, D), p0),
            pl.BlockSpec((TB, DG), p0),
            full((H0, D)), full((H0, D)), full((H0, DG)),
            full((1, H0)), full((1, H0)), full((1, H0)),
            full((H1, H0)), full((1, H1)), full((1, H1)), full((1, H1)),
            full((H2, H1)), full((1, H2)), full((1, H2)), full((1, H2)),
            full((1, H2)), full((1, 1)),
        ],
        out_specs=pl.BlockSpec((1, 1, B), lambda s: (0, 0, 0)),
        out_shape=jax.ShapeDtypeStruct((1, 1, B), f32),
        scratch_shapes=[
            pltpu.VMEM((B, H0), f32),
            pltpu.VMEM((1, H0), f32), pltpu.VMEM((1, H0), f32),
        ],
    )(ue, ie, gemb,
      W0[:, :D], W0[:, D:2 * D], W0[:, 2 * D:],
      row(b0), row(gamma0), row(beta0),
      W1, row(b1), row(gamma1), row(beta1),
      W2, row(b2), row(gamma2), row(beta2),
      Wp, row(bp))

    return out.reshape(B)


# speedup vs baseline: 11.1461x; 1.0938x over previous
import functools

import jax
import jax.numpy as jnp
from jax import lax
from jax.experimental import pallas as pl
from jax.experimental.pallas import tpu as pltpu
from jax.experimental.pallas import tpu_sc as plsc

_EPS = 1e-5

_NC = 2
_NS = 16
_NW = _NC * _NS
_SUB = 128


def _sc_gather(user_ids, item_ids, user_table, item_table):
    B = user_ids.shape[0]
    D = user_table.shape[1]
    ch = B // _NW
    nchunk = ch // _SUB

    uids2 = user_ids.reshape(B // _SUB, _SUB)
    iids2 = item_ids.reshape(B // _SUB, _SUB)

    mesh = plsc.VectorSubcoreMesh(core_axis_name="c", subcore_axis_name="s")

    @functools.partial(
        pl.kernel,
        mesh=mesh,
        out_type=jax.ShapeDtypeStruct((B, 2 * D), jnp.float32),
        scratch_types=[
            pltpu.VMEM((nchunk, _SUB), jnp.int32),
            pltpu.VMEM((ch, D), jnp.float32),
            pltpu.SemaphoreType.DMA,
        ],
    )
    def gk(uids, iids, ut, it, out, idx_v, rows_v, sem):
        wid = lax.axis_index("s") * _NC + lax.axis_index("c")
        base = wid * ch
        for col, ids, tbl in ((0, uids, ut), (D, iids, it)):
            pltpu.sync_copy(ids.at[pl.ds(wid * nchunk, nchunk)], idx_v)
            handles = []
            for j in range(nchunk):
                handles.append(
                    pltpu.async_copy(
                        tbl.at[idx_v.at[j]],
                        rows_v.at[pl.ds(j * _SUB, _SUB)],
                        sem,
                    )
                )
            for h in handles:
                h.wait()
            pltpu.sync_copy(rows_v, out.at[pl.ds(base, ch), pl.ds(col, D)])

    return gk(uids2, iids2, user_table, item_table)


def _genre_body(gid_ref, gt_ref, aw_ref, ge_ref, *, G, NGP, TB):
    f32 = jnp.float32
    bf16 = jnp.bfloat16
    L = G * NGP
    gt = gt_ref[...]
    aw = aw_ref[...]
    lt_row = lax.dot_general(aw, gt, (((1,), (1,)), ((), ())),
                             preferred_element_type=f32)
    lt_tiled = jnp.concatenate([lt_row] * G, axis=1)
    rg = lax.broadcasted_iota(jnp.int32, (G, L), 0)
    rl = lax.broadcasted_iota(jnp.int32, (G, L), 1)
    R = (rg == rl // NGP).astype(bf16)
    gid_rep = lax.dot_general(gid_ref[...].astype(bf16), R,
                              (((1,), (0,)), ((), ())),
                              preferred_element_type=f32)
    lane_t = (lax.broadcasted_iota(jnp.int32, (TB, L), 1) % NGP).astype(f32)
    cmp = gid_rep == lane_t
    mlt = jnp.where(cmp, lt_tiled, 0.0)
    l5 = lax.dot_general(mlt.astype(bf16), R, (((1,), (1,)), ((), ())),
                         preferred_element_type=f32)
    m = jnp.max(l5, axis=1, keepdims=True)
    e5 = jnp.exp(l5 - m)
    w5 = e5 / jnp.sum(e5, axis=1, keepdims=True)
    w_rep = lax.dot_general(w5.astype(bf16), R, (((1,), (0,)), ((), ())),
                            preferred_element_type=f32)
    woh = jnp.where(cmp, w_rep, 0.0)
    gt5 = jnp.concatenate([gt] * G, axis=0).astype(bf16)
    ge_ref[...] = lax.dot_general(woh.astype(bf16), gt5,
                                  (((1,), (0,)), ((), ())),
                                  preferred_element_type=f32).astype(bf16)


def _mlp_body(uei_ref, gemb_ref,
              w0ui_ref, w0g_ref, b0_ref, g0_ref, be0_ref,
              w1_ref, b1_ref, g1_ref, be1_ref,
              w2_ref, b2_ref, g2_ref, be2_ref,
              wp_ref, bp_ref,
              o_ref,
              z0_sc, s0_sc, ss0_sc,
              *, B, TB, NT):
    s = pl.program_id(0)
    bf16 = jnp.bfloat16

    def mm(a, w_ref):
        return lax.dot_general(a.astype(bf16), w_ref[...].astype(bf16),
                               (((1,), (1,)), ((), ())),
                               preferred_element_type=jnp.float32)

    @pl.when(s < NT)
    def _pass0():
        row = pl.ds(pl.multiple_of(s * TB, TB), TB)
        z = mm(uei_ref[...], w0ui_ref)
        z = z + mm(gemb_ref[...], w0g_ref) + b0_ref[...]
        z0_sc[row, :] = z

        @pl.when(s == 0)
        def _():
            s0_sc[...] = jnp.zeros_like(s0_sc)
            ss0_sc[...] = jnp.zeros_like(ss0_sc)

        s0_sc[...] += jnp.sum(z, axis=0, keepdims=True)
        ss0_sc[...] += jnp.sum(z * z, axis=0, keepdims=True)

    def bn_relu(z, s_row, ss_row, g_ref, be_ref):
        mean = s_row / B
        var = ss_row / B - mean * mean
        a = lax.rsqrt(var + _EPS) * g_ref[...]
        c = be_ref[...] - mean * a
        return jnp.maximum(z * a + c, 0.0)

    @pl.when(s == NT)
    def _rest():
        h0 = bn_relu(z0_sc[...], s0_sc[...], ss0_sc[...], g0_ref, be0_ref)
        z1 = mm(h0, w1_ref) + b1_ref[...]
        h1 = bn_relu(z1, jnp.sum(z1, axis=0, keepdims=True),
                     jnp.sum(z1 * z1, axis=0, keepdims=True),
                     g1_ref, be1_ref)
        z2 = mm(h1, w2_ref) + b2_ref[...]
        h2 = bn_relu(z2, jnp.sum(z2, axis=0, keepdims=True),
                     jnp.sum(z2 * z2, axis=0, keepdims=True),
                     g2_ref, be2_ref)
        ot = lax.dot_general(wp_ref[...].astype(bf16), h2.astype(bf16),
                             (((1,), (1,)), ((), ())),
                             preferred_element_type=jnp.float32) + bp_ref[0, 0]
        o_ref[...] = (jax.nn.sigmoid(ot) * 5.0).reshape(1, 1, B)


def kernel(user_ids, item_ids, genre_ids, user_table, item_table, genre_table,
           attn_w, attn_b, W0, b0, gamma0, beta0, W1, b1, gamma1, beta1,
           W2, b2, gamma2, beta2, Wp, bp):
    B = user_ids.shape[0]
    G = genre_ids.shape[1]
    NG, DG = genre_table.shape
    D = user_table.shape[1]
    TB = 4096
    nt = B // TB
    f32 = jnp.float32

    uei = _sc_gather(user_ids.astype(jnp.int32), item_ids.astype(jnp.int32),
                     user_table, item_table)

    NGP = 24
    gt_pad = jnp.zeros((NGP, DG), f32).at[:NG].set(genre_table)
    gid = genre_ids.astype(jnp.int32)

    gemb = pl.pallas_call(
        functools.partial(_genre_body, G=G, NGP=NGP, TB=TB),
        grid=(nt,),
        in_specs=[
            pl.BlockSpec((TB, G), lambda i: (i, 0)),
            pl.BlockSpec((NGP, DG), lambda i: (0, 0)),
            pl.BlockSpec((1, DG), lambda i: (0, 0)),
        ],
        out_specs=pl.BlockSpec((TB, DG), lambda i: (i, 0)),
        out_shape=jax.ShapeDtypeStruct((B, DG), jnp.bfloat16),
    )(gid, gt_pad, attn_w)

    H0, H1, H2 = W0.shape[0], W1.shape[0], W2.shape[0]
    row = lambda v: v.reshape(1, -1)

    p0 = lambda s: (jnp.minimum(s, nt - 1), 0)
    fix = lambda s: (0, 0)
    full = lambda sh: pl.BlockSpec(sh, fix)

    out = pl.pallas_call(
        functools.partial(_mlp_body, B=B, TB=TB, NT=nt),
        grid=(nt + 1,),
        in_specs=[
            pl.BlockSpec((TB, 2 * D), p0),
            pl.BlockSpec((TB, DG), p0),
            full((H0, 2 * D)), full((H0, DG)),
            full((1, H0)), full((1, H0)), full((1, H0)),
            full((H1, H0)), full((1, H1)), full((1, H1)), full((1, H1)),
            full((H2, H1)), full((1, H2)), full((1, H2)), full((1, H2)),
            full((1, H2)), full((1, 1)),
        ],
        out_specs=pl.BlockSpec((1, 1, B), lambda s: (0, 0, 0)),
        out_shape=jax.ShapeDtypeStruct((1, 1, B), f32),
        scratch_shapes=[
            pltpu.VMEM((B, H0), f32),
            pltpu.VMEM((1, H0), f32), pltpu.VMEM((1, H0), f32),
        ],
    )(uei, gemb,
      W0[:, :2 * D], W0[:, 2 * D:],
      row(b0), row(gamma0), row(beta0),
      W1, row(b1), row(gamma1), row(beta1),
      W2, row(b2), row(gamma2), row(beta2),
      Wp, row(bp))

    return out.reshape(B)
